# baseline (speedup 1.0000x reference)
import jax
import jax.numpy as jnp
from jax.experimental import pallas as pl
from jax.experimental.pallas import tpu as pltpu

K = 65536
DIM = 128
B = 4096
NC = K // B
VB = B // DIM
VK = K // DIM


def _copy_kernel(s_ref, f_ref, k_ref, v_ref, kv_ref, of_ref, ov_ref, op_ref,
                 buf, vbuf, kvbuf, insem, outsem):
    p0 = s_ref[0] // B
    op_ref[0] = (s_ref[0] + B) % K

    def in_desc(c):
        return pltpu.make_async_copy(
            f_ref.at[pl.ds(c * B, B), :], buf.at[c], insem)

    def out_desc(c):
        return pltpu.make_async_copy(
            buf.at[c], of_ref.at[pl.ds(c * B, B), :], outsem)

    in_v = pltpu.make_async_copy(v_ref, vbuf, insem)
    in_kv = pltpu.make_async_copy(kv_ref, kvbuf, insem)
    in_v.start()
    in_kv.start()

    for c in range(NC):
        @pl.when(c != p0)
        def _(c=c):
            in_desc(c).start()

        @pl.when(c == p0)
        def _(c=c):
            pltpu.make_async_copy(k_ref, buf.at[c], insem).start()

    in_v.wait()
    in_kv.wait()
    for c in range(NC):
        in_desc(c).wait()

    vbuf[pl.ds(p0 * VB, VB), :] = kvbuf[...]
    out_v = pltpu.make_async_copy(vbuf, ov_ref, outsem)
    out_v.start()
    for c in range(NC):
        out_desc(c).start()

    out_v.wait()
    for c in range(NC):
        out_desc(c).wait()


def kernel(features, vids, keys, key_vids, ptr):
    ptr_arr = jnp.atleast_1d(jnp.asarray(ptr, dtype=jnp.int32))
    vids2d = vids.reshape(VK, DIM)
    kv2d = key_vids.reshape(VB, DIM)

    features_new, vids_new2d, new_ptr_arr = pl.pallas_call(
        _copy_kernel,
        in_specs=[
            pl.BlockSpec(memory_space=pltpu.SMEM),
            pl.BlockSpec(memory_space=pl.MemorySpace.ANY),
            pl.BlockSpec(memory_space=pl.MemorySpace.ANY),
            pl.BlockSpec(memory_space=pl.MemorySpace.ANY),
            pl.BlockSpec(memory_space=pl.MemorySpace.ANY),
        ],
        out_specs=[
            pl.BlockSpec(memory_space=pl.MemorySpace.ANY),
            pl.BlockSpec(memory_space=pl.MemorySpace.ANY),
            pl.BlockSpec(memory_space=pltpu.SMEM),
        ],
        scratch_shapes=[
            pltpu.VMEM((NC, B, DIM), jnp.float32),
            pltpu.VMEM((VK, DIM), jnp.float32),
            pltpu.VMEM((VB, DIM), jnp.float32),
            pltpu.SemaphoreType.DMA,
            pltpu.SemaphoreType.DMA,
        ],
        out_shape=[
            jax.ShapeDtypeStruct((K, DIM), features.dtype),
            jax.ShapeDtypeStruct((VK, DIM), vids.dtype),
            jax.ShapeDtypeStruct((1,), jnp.int32),
        ],
    )(ptr_arr, features, keys, vids2d, kv2d)

    return features_new, vids_new2d.reshape(K), new_ptr_arr.reshape(())


# speedup vs baseline: 1.0355x; 1.0355x over previous
import jax
import jax.numpy as jnp
from jax.experimental import pallas as pl
from jax.experimental.pallas import tpu as pltpu

K = 65536
DIM = 128
B = 4096
NC = K // B
VB = B // DIM
VK = K // DIM


def _copy_kernel(s_ref, f_ref, k_ref, v_ref, kv_ref, of_ref, ov_ref,
                 buf, vbuf, kvbuf, insem, outsem):
    p0 = s_ref[0] // B

    def in_desc(c):
        return pltpu.make_async_copy(
            f_ref.at[pl.ds(c * B, B), :], buf.at[c], insem)

    def out_desc(c):
        return pltpu.make_async_copy(
            buf.at[c], of_ref.at[pl.ds(c * B, B), :], outsem)

    in_v = pltpu.make_async_copy(v_ref, vbuf, insem)
    in_kv = pltpu.make_async_copy(kv_ref, kvbuf, insem)
    in_v.start()
    in_kv.start()

    for c in range(NC):
        @pl.when(c != p0)
        def _(c=c):
            in_desc(c).start()

        @pl.when(c == p0)
        def _(c=c):
            pltpu.make_async_copy(k_ref, buf.at[c], insem).start()

    in_v.wait()
    in_kv.wait()
    for c in range(NC):
        in_desc(c).wait()

    vbuf[pl.ds(p0 * VB, VB), :] = kvbuf[...]
    out_v = pltpu.make_async_copy(vbuf, ov_ref, outsem)
    out_v.start()
    for c in range(NC):
        out_desc(c).start()

    out_v.wait()
    for c in range(NC):
        out_desc(c).wait()


def kernel(features, vids, keys, key_vids, ptr):
    ptr_arr = jnp.atleast_1d(jnp.asarray(ptr, dtype=jnp.int32))
    vids2d = vids.reshape(VK, DIM)
    kv2d = key_vids.reshape(VB, DIM)

    grid_spec = pltpu.PrefetchScalarGridSpec(
        num_scalar_prefetch=1,
        grid=(1,),
        in_specs=[
            pl.BlockSpec(memory_space=pl.MemorySpace.ANY),
            pl.BlockSpec(memory_space=pl.MemorySpace.ANY),
            pl.BlockSpec(memory_space=pl.MemorySpace.ANY),
            pl.BlockSpec(memory_space=pl.MemorySpace.ANY),
        ],
        out_specs=[
            pl.BlockSpec(memory_space=pl.MemorySpace.ANY),
            pl.BlockSpec(memory_space=pl.MemorySpace.ANY),
        ],
        scratch_shapes=[
            pltpu.VMEM((NC, B, DIM), jnp.float32),
            pltpu.VMEM((VK, DIM), jnp.float32),
            pltpu.VMEM((VB, DIM), jnp.float32),
            pltpu.SemaphoreType.DMA,
            pltpu.SemaphoreType.DMA,
        ],
    )

    features_new, vids_new2d = pl.pallas_call(
        _copy_kernel,
        grid_spec=grid_spec,
        out_shape=[
            jax.ShapeDtypeStruct((K, DIM), features.dtype),
            jax.ShapeDtypeStruct((VK, DIM), vids.dtype),
        ],
    )(ptr_arr, features, keys, vids2d, kv2d)

    new_ptr = ((ptr_arr[0] + B) % K).astype(jnp.int32)
    return features_new, vids_new2d.reshape(K), new_ptr
